# baseline (speedup 1.0000x reference)
import jax
import jax.numpy as jnp
from jax.experimental import pallas as pl

K = 30
OUT_CHANNELS = 50
B, P = 16, 2048


def _mlp_bn(h, layers):
    for (W, b, g, be) in layers:
        h = jnp.maximum(h @ W + b, 0.0)
        h = g * (h / jnp.sqrt(1.0 + 1e-5)) + be
    return h


def _knn_idx(x):
    sq = jnp.sum(x * x, axis=-1)
    d2 = sq[:, :, None] + sq[:, None, :] - 2.0 * jnp.einsum('bpd,bqd->bpq', x, x)
    _, idx = jax.lax.top_k(-d2, K)
    return idx


def _edge_conv(x, layers):
    idx = _knn_idx(x)
    xj = jax.vmap(lambda xb, ib: xb[ib])(x, idx)
    xi = jnp.broadcast_to(x[:, :, None, :], xj.shape)
    msg = jnp.concatenate([xi, xj - xi], axis=-1)
    h = _mlp_bn(msg, layers)
    return jnp.max(h, axis=2)


def kernel(x, pos, params, batch):
    x0 = jnp.concatenate([x, pos], axis=-1).reshape(B, P, 6)
    x1 = _edge_conv(x0, params['conv1'])
    x2 = _edge_conv(x1, params['conv2'])
    x3 = _edge_conv(x2, params['conv3'])
    h = jnp.concatenate([x1, x2, x3], axis=-1)
    n_layers = len(params['mlp'])
    for i, (W, b) in enumerate(params['mlp']):
        h = h @ W + b
        if i < n_layers - 1:
            h = jnp.maximum(h, 0.0)
    out = h.reshape(B * P, OUT_CHANNELS)
    return jax.nn.log_softmax(out, axis=1)


# speedup vs baseline: 6.3567x; 6.3567x over previous
import math

import jax
import jax.numpy as jnp
from jax import lax
from jax.experimental import pallas as pl
from jax.experimental.pallas import tpu as pltpu
from jax.experimental.pallas import tpu_sc as plsc

K = 30
KP = 32
OUT_CHANNELS = 50
B, P = 16, 2048
N = B * P
RT = 256
TPC = P // RT
FT = 512
INV_S = 1.0 / math.sqrt(1.0 + 1e-5)

NC, NS = 2, 16
NW = NC * NS
E = N * KP
PER_W = E // NW
IW = 128
SUBG = 4
GR = SUBG * IW
NG = PER_W // GR
NROW = PER_W // IW


def _knn_body(xr_ref, xf_ref, wd_ref, wb_ref, b1_ref, idx_ref, a_ref, bf_ref):
    xf = xf_ref[...]
    ss = jnp.sum(xf * xf, axis=1, keepdims=True)
    y = jnp.concatenate([xf, ss], axis=1)
    xr = xr_ref[...]
    zr = jnp.concatenate([-2.0 * xr, jnp.ones((RT, 1), jnp.float32)], axis=1)
    d2 = lax.dot_general(zr, y, (((1,), (1,)), ((), ())),
                         preferred_element_type=jnp.float32)
    col = lax.broadcasted_iota(jnp.int32, (RT, P), 1)
    kcol = lax.broadcasted_iota(jnp.int32, (RT, KP), 1)
    inf = jnp.float32(jnp.inf)

    def step(k, carry):
        d2m, acc = carry
        m = jnp.min(d2m, axis=1, keepdims=True)
        am = jnp.min(jnp.where(d2m <= m, col, P), axis=1, keepdims=True)
        acc = jnp.where(kcol == k, am, acc)
        d2m = jnp.where(col == am, inf, d2m)
        return d2m, acc

    _, acc = lax.fori_loop(0, K, step, (d2, jnp.zeros((RT, KP), jnp.int32)))
    acc = jnp.where(kcol >= K, acc[:, 0:1], acc)
    base = (pl.program_id(0) // TPC) * P
    idx_ref[...] = acc + base

    a_ref[...] = jnp.dot(xr, wd_ref[...],
                         preferred_element_type=jnp.float32) + b1_ref[...]
    bf_ref[...] = jnp.dot(xr, wb_ref[...], preferred_element_type=jnp.float32)


def _knn_pre(x, wd, wb, b1):
    d = x.shape[1]
    return pl.pallas_call(
        _knn_body,
        grid=(N // RT,),
        in_specs=[
            pl.BlockSpec((RT, d), lambda t: (t, 0)),
            pl.BlockSpec((P, d), lambda t: (t // TPC, 0)),
            pl.BlockSpec((d, 64), lambda t: (0, 0)),
            pl.BlockSpec((d, 64), lambda t: (0, 0)),
            pl.BlockSpec((1, 64), lambda t: (0, 0)),
        ],
        out_specs=[
            pl.BlockSpec((RT, KP), lambda t: (t, 0)),
            pl.BlockSpec((RT, 64), lambda t: (t, 0)),
            pl.BlockSpec((RT, 64), lambda t: (t, 0)),
        ],
        out_shape=[
            jax.ShapeDtypeStruct((N, KP), jnp.int32),
            jax.ShapeDtypeStruct((N, 64), jnp.float32),
            jax.ShapeDtypeStruct((N, 64), jnp.float32),
        ],
    )(x, x, wd, wb, b1)


def _gather_body(tbl_hbm, idx_hbm, out_hbm, idx_v, rows_a, rows_b, sem_a, sem_b):
    cid = lax.axis_index("c")
    sid = lax.axis_index("s")
    wid = sid * NC + cid
    pltpu.sync_copy(idx_hbm.at[wid], idx_v)

    def fire(g, buf, sem):
        for j in range(SUBG):
            pltpu.async_copy(tbl_hbm.at[idx_v.at[g * SUBG + j]],
                             buf.at[pl.ds(j * IW, IW)], sem)

    def drain(g, buf, sem):
        for j in range(SUBG):
            pltpu.make_async_copy(tbl_hbm.at[idx_v.at[g * SUBG + j]],
                                  buf.at[pl.ds(j * IW, IW)], sem).wait()

    fire(0, rows_a, sem_a)

    def body(t, carry):
        g0 = 2 * t
        g1 = g0 + 1
        fire(g1, rows_b, sem_b)
        drain(g0, rows_a, sem_a)
        pltpu.sync_copy(rows_a, out_hbm.at[wid, g0])

        @pl.when(t < (NG // 2 - 1))
        def _():
            fire(g0 + 2, rows_a, sem_a)

        drain(g1, rows_b, sem_b)
        pltpu.sync_copy(rows_b, out_hbm.at[wid, g1])
        return carry

    lax.fori_loop(0, NG // 2, body, 0)


def _sc_gather(tbl, idx_flat):
    idx3 = idx_flat.reshape(NW, NROW, IW)
    mesh = plsc.VectorSubcoreMesh(core_axis_name="c", subcore_axis_name="s")
    out = pl.kernel(
        _gather_body,
        out_type=jax.ShapeDtypeStruct((NW, NG, GR, 64), jnp.float32),
        mesh=mesh,
        scratch_types=[
            pltpu.VMEM((NROW, IW), jnp.int32),
            pltpu.VMEM((GR, 64), jnp.float32),
            pltpu.VMEM((GR, 64), jnp.float32),
            pltpu.SemaphoreType.DMA,
            pltpu.SemaphoreType.DMA,
        ],
        compiler_params=pltpu.CompilerParams(use_tc_tiling_on_sc=False),
    )(tbl, idx3)
    return out.reshape(N, KP, 64)


def _agg_body(a_ref, g_ref, w2_ref, b2_ref, s2_ref, be2_ref, out_ref):
    a = a_ref[...]
    g = g_ref[...]
    h1 = jnp.maximum(a[:, None, :] + g, 0.0).reshape(RT * KP, 64)
    h2 = jnp.maximum(
        jnp.dot(h1, w2_ref[...], preferred_element_type=jnp.float32)
        + b2_ref[...], 0.0)
    a2 = h2 * s2_ref[...] + be2_ref[...]
    out_ref[...] = jnp.max(a2.reshape(RT, KP, 64), axis=1)


def _edge_agg(a, g3, w2, b2, s2, be2):
    return pl.pallas_call(
        _agg_body,
        grid=(N // RT,),
        in_specs=[
            pl.BlockSpec((RT, 64), lambda t: (t, 0)),
            pl.BlockSpec((RT, KP, 64), lambda t: (t, 0, 0)),
            pl.BlockSpec((64, 64), lambda t: (0, 0)),
            pl.BlockSpec((1, 64), lambda t: (0, 0)),
            pl.BlockSpec((1, 64), lambda t: (0, 0)),
            pl.BlockSpec((1, 64), lambda t: (0, 0)),
        ],
        out_specs=pl.BlockSpec((RT, 64), lambda t: (t, 0)),
        out_shape=jax.ShapeDtypeStruct((N, 64), jnp.float32),
    )(a, g3, w2, b2, s2, be2)


def _final_body(x1_ref, x2_ref, x3_ref, w0, b0, w1, b1, w2, b2, w3, b3, out_ref):
    h = jnp.concatenate([x1_ref[...], x2_ref[...], x3_ref[...]], axis=1)
    h = jnp.maximum(jnp.dot(h, w0[...], preferred_element_type=jnp.float32) + b0[...], 0.0)
    h = jnp.maximum(jnp.dot(h, w1[...], preferred_element_type=jnp.float32) + b1[...], 0.0)
    h = jnp.maximum(jnp.dot(h, w2[...], preferred_element_type=jnp.float32) + b2[...], 0.0)
    h = jnp.dot(h, w3[...], preferred_element_type=jnp.float32) + b3[...]
    m = jnp.max(h, axis=1, keepdims=True)
    e = h - m
    lse = jnp.log(jnp.sum(jnp.exp(e), axis=1, keepdims=True))
    out_ref[...] = e - lse


def _final_mlp(x1, x2, x3, mlp):
    (w0, b0), (w1, b1), (w2, b2), (w3, b3) = mlp
    args = [x1, x2, x3,
            w0, b0.reshape(1, -1), w1, b1.reshape(1, -1),
            w2, b2.reshape(1, -1), w3, b3.reshape(1, -1)]
    in_specs = [pl.BlockSpec((FT, 64), lambda t: (t, 0))] * 3
    for wgt, bia in ((w0, b0), (w1, b1), (w2, b2), (w3, b3)):
        in_specs.append(pl.BlockSpec(wgt.shape, lambda t: (0, 0)))
        in_specs.append(pl.BlockSpec((1, bia.shape[0]), lambda t: (0, 0)))
    return pl.pallas_call(
        _final_body,
        grid=(N // FT,),
        in_specs=in_specs,
        out_specs=pl.BlockSpec((FT, OUT_CHANNELS), lambda t: (t, 0)),
        out_shape=jax.ShapeDtypeStruct((N, OUT_CHANNELS), jnp.float32),
    )(*args)


def _fold_conv(layers):
    (w1, b1, g1, be1), (w2, b2, g2, be2) = layers
    d = w1.shape[0] // 2
    wtop, wbot = w1[:d], w1[d:]
    wd = wtop - wbot
    s1 = g1 * INV_S
    w2p = s1[:, None] * w2
    b2p = be1 @ w2 + b2
    s2 = g2 * INV_S
    return (wd, wbot, b1.reshape(1, 64), w2p, b2p.reshape(1, 64),
            s2.reshape(1, 64), be2.reshape(1, 64))


def _conv(x, fold):
    wd, wb, b1, w2p, b2p, s2, be2 = fold
    idx, a, bf = _knn_pre(x, wd, wb, b1)
    g3 = _sc_gather(bf, idx.reshape(E))
    return _edge_agg(a, g3, w2p, b2p, s2, be2)


def kernel(x, pos, params, batch):
    x0 = jnp.concatenate([x, pos], axis=1)
    x1 = _conv(x0, _fold_conv(params['conv1']))
    x2 = _conv(x1, _fold_conv(params['conv2']))
    x3 = _conv(x2, _fold_conv(params['conv3']))
    return _final_mlp(x1, x2, x3, params['mlp'])


# speedup vs baseline: 7.1521x; 1.1251x over previous
import math

import jax
import jax.numpy as jnp
from jax import lax
from jax.experimental import pallas as pl
from jax.experimental.pallas import tpu as pltpu
from jax.experimental.pallas import tpu_sc as plsc

K = 30
KP = 32
OUT_CHANNELS = 50
B, P = 16, 2048
N = B * P
RT = 256
TPC = P // RT
FT = 512
INV_S = 1.0 / math.sqrt(1.0 + 1e-5)

NC, NS = 2, 16
NW = NC * NS
E = N * KP
PER_W = E // NW
IW = 128
SUBG = 4
GR = SUBG * IW
NG = PER_W // GR
NROW = PER_W // IW


CT = 6
CC = 128
CG = P // CC
NCAND = CT * CC


def _full_extract(d2, kcol):
    col = lax.broadcasted_iota(jnp.int32, (RT, P), 1)
    inf = jnp.float32(jnp.inf)

    def step(k, carry):
        d2m, acc = carry
        m = jnp.min(d2m, axis=1, keepdims=True)
        am = jnp.min(jnp.where(d2m <= m, col, P), axis=1, keepdims=True)
        acc = jnp.where(kcol == k, am, acc)
        d2m = jnp.where(col == am, inf, d2m)
        return d2m, acc

    _, acc = lax.fori_loop(0, K, step, (d2, jnp.zeros((RT, KP), jnp.int32)))
    return acc


def _knn_body(xr_ref, xf_ref, wd_ref, wb_ref, b1_ref, idx_ref, a_ref, bf_ref):
    xf = xf_ref[...]
    ss = jnp.sum(xf * xf, axis=1, keepdims=True)
    y = jnp.concatenate([xf, ss], axis=1)
    xr = xr_ref[...]
    zr = jnp.concatenate([-2.0 * xr, jnp.ones((RT, 1), jnp.float32)], axis=1)
    d2 = lax.dot_general(zr, y, (((1,), (1,)), ((), ())),
                         preferred_element_type=jnp.float32)
    kcol = lax.broadcasted_iota(jnp.int32, (RT, KP), 1)
    inf = jnp.float32(jnp.inf)

    d2v = d2.reshape(RT, CG, CC)
    giota = lax.broadcasted_iota(jnp.int32, (RT, CG, CC), 1)
    vals, gsel = [], []
    for _t in range(CT):
        mc = jnp.min(d2v, axis=1, keepdims=True)
        amg = jnp.min(jnp.where(d2v <= mc, giota, CG), axis=1,
                      keepdims=True)
        vals.append(mc)
        gsel.append(amg)
        d2v = jnp.where(giota == amg, inf, d2v)
    cv = jnp.concatenate(vals, axis=1).reshape(RT, NCAND)
    cg = jnp.concatenate(gsel, axis=1).reshape(RT, NCAND)
    labs = lax.broadcasted_iota(jnp.int32, (RT, CT, CC), 2).reshape(RT, NCAND)
    cc_abs = cg * CC + labs

    siota = lax.broadcasted_iota(jnp.int32, (RT, NCAND), 1)

    def step(k, carry):
        cvm, acc = carry
        m = jnp.min(cvm, axis=1, keepdims=True)
        slot = jnp.min(jnp.where(cvm <= m, siota, NCAND), axis=1,
                       keepdims=True)
        hit = siota == slot
        am = jnp.sum(jnp.where(hit, cc_abs, 0), axis=1, keepdims=True)
        acc = jnp.where(kcol == k, am, acc)
        cvm = jnp.where(hit, inf, cvm)
        return cvm, acc

    cvm, acc = lax.fori_loop(0, K, step,
                             (cv, jnp.zeros((RT, KP), jnp.int32)))

    exhausted = jnp.all(jnp.isinf(cvm.reshape(RT, CT, CC)), axis=1)
    need_exact = jnp.any(jnp.sum(jnp.where(exhausted, 1, 0)) > 0)
    acc = lax.cond(need_exact, lambda: _full_extract(d2, kcol), lambda: acc)

    acc = jnp.where(kcol >= K, acc[:, 0:1], acc)
    base = (pl.program_id(0) // TPC) * P
    idx_ref[...] = acc + base

    a_ref[...] = jnp.dot(xr, wd_ref[...],
                         preferred_element_type=jnp.float32) + b1_ref[...]
    bf_ref[...] = jnp.dot(xr, wb_ref[...], preferred_element_type=jnp.float32)


def _knn_pre(x, wd, wb, b1):
    d = x.shape[1]
    return pl.pallas_call(
        _knn_body,
        grid=(N // RT,),
        in_specs=[
            pl.BlockSpec((RT, d), lambda t: (t, 0)),
            pl.BlockSpec((P, d), lambda t: (t // TPC, 0)),
            pl.BlockSpec((d, 64), lambda t: (0, 0)),
            pl.BlockSpec((d, 64), lambda t: (0, 0)),
            pl.BlockSpec((1, 64), lambda t: (0, 0)),
        ],
        out_specs=[
            pl.BlockSpec((RT, KP), lambda t: (t, 0)),
            pl.BlockSpec((RT, 64), lambda t: (t, 0)),
            pl.BlockSpec((RT, 64), lambda t: (t, 0)),
        ],
        out_shape=[
            jax.ShapeDtypeStruct((N, KP), jnp.int32),
            jax.ShapeDtypeStruct((N, 64), jnp.float32),
            jax.ShapeDtypeStruct((N, 64), jnp.float32),
        ],
    )(x, x, wd, wb, b1)


def _gather_body(tbl_hbm, idx_hbm, out_hbm, idx_v, rows_a, rows_b, sem_a, sem_b):
    cid = lax.axis_index("c")
    sid = lax.axis_index("s")
    wid = sid * NC + cid
    pltpu.sync_copy(idx_hbm.at[wid], idx_v)

    def fire(g, buf, sem):
        for j in range(SUBG):
            pltpu.async_copy(tbl_hbm.at[idx_v.at[g * SUBG + j]],
                             buf.at[pl.ds(j * IW, IW)], sem)

    def drain(g, buf, sem):
        for j in range(SUBG):
            pltpu.make_async_copy(tbl_hbm.at[idx_v.at[g * SUBG + j]],
                                  buf.at[pl.ds(j * IW, IW)], sem).wait()

    fire(0, rows_a, sem_a)

    def body(t, carry):
        g0 = 2 * t
        g1 = g0 + 1
        fire(g1, rows_b, sem_b)
        drain(g0, rows_a, sem_a)
        pltpu.sync_copy(rows_a, out_hbm.at[wid, g0])

        @pl.when(t < (NG // 2 - 1))
        def _():
            fire(g0 + 2, rows_a, sem_a)

        drain(g1, rows_b, sem_b)
        pltpu.sync_copy(rows_b, out_hbm.at[wid, g1])
        return carry

    lax.fori_loop(0, NG // 2, body, 0)


def _sc_gather(tbl, idx_flat):
    idx3 = idx_flat.reshape(NW, NROW, IW)
    mesh = plsc.VectorSubcoreMesh(core_axis_name="c", subcore_axis_name="s")
    out = pl.kernel(
        _gather_body,
        out_type=jax.ShapeDtypeStruct((NW, NG, GR, 64), jnp.float32),
        mesh=mesh,
        scratch_types=[
            pltpu.VMEM((NROW, IW), jnp.int32),
            pltpu.VMEM((GR, 64), jnp.float32),
            pltpu.VMEM((GR, 64), jnp.float32),
            pltpu.SemaphoreType.DMA,
            pltpu.SemaphoreType.DMA,
        ],
        compiler_params=pltpu.CompilerParams(use_tc_tiling_on_sc=False),
    )(tbl, idx3)
    return out.reshape(N, KP, 64)


def _agg_body(a_ref, g_ref, w2_ref, b2_ref, s2_ref, be2_ref, out_ref):
    a = a_ref[...]
    g = g_ref[...]
    h1 = jnp.maximum(a[:, None, :] + g, 0.0).reshape(RT * KP, 64)
    h2 = jnp.maximum(
        jnp.dot(h1, w2_ref[...], preferred_element_type=jnp.float32)
        + b2_ref[...], 0.0)
    a2 = h2 * s2_ref[...] + be2_ref[...]
    out_ref[...] = jnp.max(a2.reshape(RT, KP, 64), axis=1)


def _edge_agg(a, g3, w2, b2, s2, be2):
    return pl.pallas_call(
        _agg_body,
        grid=(N // RT,),
        in_specs=[
            pl.BlockSpec((RT, 64), lambda t: (t, 0)),
            pl.BlockSpec((RT, KP, 64), lambda t: (t, 0, 0)),
            pl.BlockSpec((64, 64), lambda t: (0, 0)),
            pl.BlockSpec((1, 64), lambda t: (0, 0)),
            pl.BlockSpec((1, 64), lambda t: (0, 0)),
            pl.BlockSpec((1, 64), lambda t: (0, 0)),
        ],
        out_specs=pl.BlockSpec((RT, 64), lambda t: (t, 0)),
        out_shape=jax.ShapeDtypeStruct((N, 64), jnp.float32),
    )(a, g3, w2, b2, s2, be2)


def _final_body(x1_ref, x2_ref, x3_ref, w0, b0, w1, b1, w2, b2, w3, b3, out_ref):
    h = jnp.concatenate([x1_ref[...], x2_ref[...], x3_ref[...]], axis=1)
    h = jnp.maximum(jnp.dot(h, w0[...], preferred_element_type=jnp.float32) + b0[...], 0.0)
    h = jnp.maximum(jnp.dot(h, w1[...], preferred_element_type=jnp.float32) + b1[...], 0.0)
    h = jnp.maximum(jnp.dot(h, w2[...], preferred_element_type=jnp.float32) + b2[...], 0.0)
    h = jnp.dot(h, w3[...], preferred_element_type=jnp.float32) + b3[...]
    m = jnp.max(h, axis=1, keepdims=True)
    e = h - m
    lse = jnp.log(jnp.sum(jnp.exp(e), axis=1, keepdims=True))
    out_ref[...] = e - lse


def _final_mlp(x1, x2, x3, mlp):
    (w0, b0), (w1, b1), (w2, b2), (w3, b3) = mlp
    args = [x1, x2, x3,
            w0, b0.reshape(1, -1), w1, b1.reshape(1, -1),
            w2, b2.reshape(1, -1), w3, b3.reshape(1, -1)]
    in_specs = [pl.BlockSpec((FT, 64), lambda t: (t, 0))] * 3
    for wgt, bia in ((w0, b0), (w1, b1), (w2, b2), (w3, b3)):
        in_specs.append(pl.BlockSpec(wgt.shape, lambda t: (0, 0)))
        in_specs.append(pl.BlockSpec((1, bia.shape[0]), lambda t: (0, 0)))
    return pl.pallas_call(
        _final_body,
        grid=(N // FT,),
        in_specs=in_specs,
        out_specs=pl.BlockSpec((FT, OUT_CHANNELS), lambda t: (t, 0)),
        out_shape=jax.ShapeDtypeStruct((N, OUT_CHANNELS), jnp.float32),
    )(*args)


def _fold_conv(layers):
    (w1, b1, g1, be1), (w2, b2, g2, be2) = layers
    d = w1.shape[0] // 2
    wtop, wbot = w1[:d], w1[d:]
    wd = wtop - wbot
    s1 = g1 * INV_S
    w2p = s1[:, None] * w2
    b2p = be1 @ w2 + b2
    s2 = g2 * INV_S
    return (wd, wbot, b1.reshape(1, 64), w2p, b2p.reshape(1, 64),
            s2.reshape(1, 64), be2.reshape(1, 64))


def _conv(x, fold):
    wd, wb, b1, w2p, b2p, s2, be2 = fold
    idx, a, bf = _knn_pre(x, wd, wb, b1)
    g3 = _sc_gather(bf, idx.reshape(E))
    return _edge_agg(a, g3, w2p, b2p, s2, be2)


def kernel(x, pos, params, batch):
    x0 = jnp.concatenate([x, pos], axis=1)
    x1 = _conv(x0, _fold_conv(params['conv1']))
    x2 = _conv(x1, _fold_conv(params['conv2']))
    x3 = _conv(x2, _fold_conv(params['conv3']))
    return _final_mlp(x1, x2, x3, params['mlp'])
